# baseline (speedup 1.0000x reference)
import functools

import jax
import jax.numpy as jnp
import numpy as np
from jax import lax
from jax.experimental import pallas as pl
from jax.experimental.pallas import tpu as pltpu
from jax.experimental.pallas import tpu_sc as plsc

_WS = 9
_PS = 3
_K = 16
_S0 = 2
_R = _WS // 2
_T, _F, _H, _W = 3, 3, 128, 128
_NH, _NW = _H // _S0, _W // _S0
_NOFF = _WS * _WS
_NPAD = 96
_NQ = _T * _NH * _NW
_NWORKERS = 32
_QB = _NQ // _NWORKERS
_BIG = 1e30


def _samp_mat():
    a = np.zeros((_NH, _H), np.float32)
    for i in range(_NH):
        for u in (2 * i - 1, 2 * i, 2 * i + 1):
            if 0 <= u < _H:
                a[i, u] = 1.0
    return a


def _samp_mat_perm():
    a = _samp_mat()
    perm = list(range(0, _NH, 2)) + list(range(1, _NH, 2))
    return a[perm]


def _dmap_body(p_ref, kp_ref, np_ref, a_ref, at_ref, dall_ref, dcross_ref):
    a = a_ref[...]
    at = at_ref[...]
    ps = [p_ref[0, c] for c in range(_F)]
    for o in range(_NOFF):
        dy, dx = o // _WS, o % _WS
        sqa = sqc = None
        for c in range(_F):
            ka = kp_ref[0, c, dy:dy + _H, dx:dx + _W]
            kn = np_ref[0, c, dy:dy + _H, dx:dx + _W]
            da = ps[c] - ka
            dn = ps[c] - kn
            sqa = da * da if sqa is None else sqa + da * da
            sqc = dn * dn if sqc is None else sqc + dn * dn
        da_ = jnp.dot(
            jnp.dot(a, sqa, preferred_element_type=jnp.float32), at,
            preferred_element_type=jnp.float32)
        dc_ = jnp.dot(
            jnp.dot(a, sqc, preferred_element_type=jnp.float32), at,
            preferred_element_type=jnp.float32)
        dall_ref[0, o, :, 0:_NW] = da_[0:_NH // 2, :]
        dall_ref[0, o, :, _NW:2 * _NW] = da_[_NH // 2:_NH, :]
        dcross_ref[0, o, :, 0:_NW] = dc_[0:_NH // 2, :]
        dcross_ref[0, o, :, _NW:2 * _NW] = dc_[_NH // 2:_NH, :]


def _dmaps(deno, noisy):
    kp = jnp.pad(deno, ((0, 0), (0, 0), (_R, _R), (_R, _R)), mode="edge")
    npd = jnp.pad(noisy, ((0, 0), (0, 0), (_R, _R), (_R, _R)), mode="edge")
    a = jnp.asarray(_samp_mat_perm())
    at = jnp.asarray(_samp_mat()).T
    hp, wp = _H + 2 * _R, _W + 2 * _R
    return pl.pallas_call(
        _dmap_body,
        grid=(_T,),
        in_specs=[
            pl.BlockSpec((1, _F, _H, _W), lambda t: (t, 0, 0, 0)),
            pl.BlockSpec((1, _F, hp, wp), lambda t: (t, 0, 0, 0)),
            pl.BlockSpec((1, _F, hp, wp), lambda t: (t, 0, 0, 0)),
            pl.BlockSpec((_NH, _H), lambda t: (0, 0)),
            pl.BlockSpec((_H, _NH), lambda t: (0, 0)),
        ],
        out_specs=[
            pl.BlockSpec((1, _NOFF, _NH // 2, 2 * _NW), lambda t: (t, 0, 0, 0)),
            pl.BlockSpec((1, _NOFF, _NH // 2, 2 * _NW), lambda t: (t, 0, 0, 0)),
        ],
        out_shape=[
            jax.ShapeDtypeStruct((_T, _NOFF, _NH // 2, 2 * _NW), jnp.float32),
            jax.ShapeDtypeStruct((_T, _NOFF, _NH // 2, 2 * _NW), jnp.float32),
        ],
    )(deno, kp, npd, a, at)


_QCHUNK = 128
_NCHUNK = _NQ // (_NWORKERS * _QCHUNK)


def _topk_body(dall_hbm, dcross_hbm, out_hbm, dall_v, dcross_v, acc_v):
    nc = 2
    wid = lax.axis_index("s") * nc + lax.axis_index("c")
    qoff = wid
    nv = _NPAD // 16
    lanes = lax.iota(jnp.int32, 16)
    offv = [jnp.minimum(16 * i + lanes, _NOFF - 1) for i in range(nv)]

    pltpu.sync_copy(dall_hbm.at[:, qoff, :],
                    dall_v.at[:, pl.ds(0, _QCHUNK)])
    pltpu.sync_copy(dcross_hbm.at[:, qoff, :],
                    dcross_v.at[:, pl.ds(0, _QCHUNK)])

    def chunk(t, acc):
        toff = t * _NOFF

        def body(q, acc):
            qv = jnp.full((16,), 0, jnp.int32) + q
            rk = rv = None
            for i in range(nv):
                rows = offv[i] + toff
                k = plsc.load_gather(dall_v, [rows, qv])
                if i == nv - 1:
                    k = jnp.where(lanes > 0, jnp.float32(_BIG), k)
                sk, sv = plsc.sort_key_val(k, rows)
                if rk is None:
                    rk, rv = sk, sv
                else:
                    skr = lax.rev(sk, (0,))
                    svr = lax.rev(sv, (0,))
                    take = rk <= skr
                    mk = jnp.where(take, rk, skr)
                    mv = jnp.where(take, rv, svr)
                    if i < nv - 1:
                        rk, rv = plsc.sort_key_val(mk, mv)
                    else:
                        rk, rv = mk, mv
            dval = plsc.load_gather(dcross_v, [rv, qv])
            msk = (rk / float(_PS * _PS * _F)) < 0.5
            return acc + jnp.where(msk, dval, jnp.float32(0.0))

        return lax.fori_loop(0, _QCHUNK, body, acc)

    acc = lax.fori_loop(0, _NCHUNK, chunk, jnp.zeros((16,), jnp.float32))
    acc_v[...] = acc
    pltpu.sync_copy(acc_v, out_hbm.at[wid])


def _topk_partials(dall, dcross):
    mesh = plsc.VectorSubcoreMesh(core_axis_name="c", subcore_axis_name="s")
    fn = functools.partial(
        pl.kernel,
        out_type=jax.ShapeDtypeStruct((_NWORKERS, 16), jnp.float32),
        mesh=mesh,
        compiler_params=pltpu.CompilerParams(needs_layout_passes=False),
        scratch_types=[
            pltpu.VMEM((_T * _NOFF, _QCHUNK + 1), jnp.float32),
            pltpu.VMEM((_T * _NOFF, _QCHUNK + 1), jnp.float32),
            pltpu.VMEM((16,), jnp.float32),
        ],
    )(_topk_body)
    return fn(dall, dcross)


def kernel(noisy, clean, deno, fflow, bflow, curr_epoch):
    d = deno[0]
    n = noisy[0]
    dall, dcross = _dmaps(d, n)
    parts = _topk_partials(dall.reshape(_T * _NOFF, _NH // 2, 2 * _NW),
                           dcross.reshape(_T * _NOFF, _NH // 2, 2 * _NW))
    return jnp.sum(parts) / jnp.float32(_NQ * _K)


# speedup vs baseline: 1.6281x; 1.6281x over previous
import functools

import jax
import jax.numpy as jnp
import numpy as np
from jax import lax
from jax.experimental import pallas as pl
from jax.experimental.pallas import tpu as pltpu
from jax.experimental.pallas import tpu_sc as plsc

_WS = 9
_PS = 3
_K = 16
_S0 = 2
_R = _WS // 2
_T, _F, _H, _W = 3, 3, 128, 128
_NH, _NW = _H // _S0, _W // _S0
_NOFF = _WS * _WS
_NPAD = 96
_NQ = _T * _NH * _NW
_NWORKERS = 32
_QB = _NQ // _NWORKERS
_BIG = 1e30


def _samp_mat():
    a = np.zeros((_NH, _H), np.float32)
    for i in range(_NH):
        for u in (2 * i - 1, 2 * i, 2 * i + 1):
            if 0 <= u < _H:
                a[i, u] = 1.0
    return a


def _dmap_body(p_ref, kp_ref, np_ref, ae_ref, ao_ref, atd_ref,
               dall_ref, dcross_ref):
    ae = ae_ref[...]
    ao = ao_ref[...]
    atd = atd_ref[...]
    lanemask = lax.broadcasted_iota(jnp.int32, (_NH // 2, 2 * _NW), 1) < _NW
    ps = [p_ref[0, c] for c in range(_F)]
    for o in range(_NOFF):
        dy, dx = o // _WS, o % _WS
        sqa = sqc = None
        for c in range(_F):
            ka = kp_ref[0, c, dy:dy + _H, dx:dx + _W]
            kn = np_ref[0, c, dy:dy + _H, dx:dx + _W]
            da = ps[c] - ka
            dn = ps[c] - kn
            sqa = da * da if sqa is None else sqa + da * da
            sqc = dn * dn if sqc is None else sqc + dn * dn
        m2a = jnp.dot(sqa, atd, preferred_element_type=jnp.float32)
        m2c = jnp.dot(sqc, atd, preferred_element_type=jnp.float32)
        dall_ref[0, o] = jnp.where(
            lanemask,
            jnp.dot(ae, m2a, preferred_element_type=jnp.float32),
            jnp.dot(ao, m2a, preferred_element_type=jnp.float32))
        dcross_ref[0, o] = jnp.where(
            lanemask,
            jnp.dot(ae, m2c, preferred_element_type=jnp.float32),
            jnp.dot(ao, m2c, preferred_element_type=jnp.float32))


def _dmaps(deno, noisy):
    kp = jnp.pad(deno, ((0, 0), (0, 0), (_R, _R), (_R, _R)), mode="edge")
    npd = jnp.pad(noisy, ((0, 0), (0, 0), (_R, _R), (_R, _R)), mode="edge")
    a = _samp_mat()
    ae = jnp.asarray(a[0::2])
    ao = jnp.asarray(a[1::2])
    atd = jnp.asarray(np.concatenate([a.T, a.T], axis=1))
    hp, wp = _H + 2 * _R, _W + 2 * _R
    return pl.pallas_call(
        _dmap_body,
        grid=(_T,),
        in_specs=[
            pl.BlockSpec((1, _F, _H, _W), lambda t: (t, 0, 0, 0)),
            pl.BlockSpec((1, _F, hp, wp), lambda t: (t, 0, 0, 0)),
            pl.BlockSpec((1, _F, hp, wp), lambda t: (t, 0, 0, 0)),
            pl.BlockSpec((_NH // 2, _H), lambda t: (0, 0)),
            pl.BlockSpec((_NH // 2, _H), lambda t: (0, 0)),
            pl.BlockSpec((_H, _H), lambda t: (0, 0)),
        ],
        out_specs=[
            pl.BlockSpec((1, _NOFF, _NH // 2, 2 * _NW), lambda t: (t, 0, 0, 0)),
            pl.BlockSpec((1, _NOFF, _NH // 2, 2 * _NW), lambda t: (t, 0, 0, 0)),
        ],
        out_shape=[
            jax.ShapeDtypeStruct((_T, _NOFF, _NH // 2, 2 * _NW), jnp.float32),
            jax.ShapeDtypeStruct((_T, _NOFF, _NH // 2, 2 * _NW), jnp.float32),
        ],
    )(deno, kp, npd, ae, ao, atd)


_QCHUNK = 128
_NCHUNK = _NQ // (_NWORKERS * _QCHUNK)


def _topk_body(dall_hbm, dcross_hbm, out_hbm, dall_v, dcross_v, acc_v):
    nc = 2
    wid = lax.axis_index("s") * nc + lax.axis_index("c")
    qoff = wid
    nv = _NPAD // 16
    lanes = lax.iota(jnp.int32, 16)
    offv = [jnp.minimum(16 * i + lanes, _NOFF - 1) for i in range(nv)]

    pltpu.sync_copy(dall_hbm.at[:, qoff, :],
                    dall_v.at[:, pl.ds(0, _QCHUNK)])
    pltpu.sync_copy(dcross_hbm.at[:, qoff, :],
                    dcross_v.at[:, pl.ds(0, _QCHUNK)])

    def chunk(t, acc):
        toff = t * _NOFF

        def body(q, acc):
            qv = jnp.full((16,), 0, jnp.int32) + q
            rk = rv = None
            for i in range(nv):
                rows = offv[i] + toff
                k = plsc.load_gather(dall_v, [rows, qv])
                if i == nv - 1:
                    k = jnp.where(lanes > 0, jnp.float32(_BIG), k)
                sk, sv = plsc.sort_key_val(k, rows)
                if rk is None:
                    rk, rv = sk, sv
                else:
                    skr = lax.rev(sk, (0,))
                    svr = lax.rev(sv, (0,))
                    take = rk <= skr
                    mk = jnp.where(take, rk, skr)
                    mv = jnp.where(take, rv, svr)
                    if i < nv - 1:
                        rk, rv = plsc.sort_key_val(mk, mv)
                    else:
                        rk, rv = mk, mv
            dval = plsc.load_gather(dcross_v, [rv, qv])
            msk = (rk / float(_PS * _PS * _F)) < 0.5
            return acc + jnp.where(msk, dval, jnp.float32(0.0))

        return lax.fori_loop(0, _QCHUNK, body, acc)

    acc = lax.fori_loop(0, _NCHUNK, chunk, jnp.zeros((16,), jnp.float32))
    acc_v[...] = acc
    pltpu.sync_copy(acc_v, out_hbm.at[wid])


def _topk_partials(dall, dcross):
    mesh = plsc.VectorSubcoreMesh(core_axis_name="c", subcore_axis_name="s")
    fn = functools.partial(
        pl.kernel,
        out_type=jax.ShapeDtypeStruct((_NWORKERS, 16), jnp.float32),
        mesh=mesh,
        compiler_params=pltpu.CompilerParams(needs_layout_passes=False),
        scratch_types=[
            pltpu.VMEM((_T * _NOFF, _QCHUNK + 1), jnp.float32),
            pltpu.VMEM((_T * _NOFF, _QCHUNK + 1), jnp.float32),
            pltpu.VMEM((16,), jnp.float32),
        ],
    )(_topk_body)
    return fn(dall, dcross)


def kernel(noisy, clean, deno, fflow, bflow, curr_epoch):
    d = deno[0]
    n = noisy[0]
    dall, dcross = _dmaps(d, n)
    parts = _topk_partials(dall.reshape(_T * _NOFF, _NH // 2, 2 * _NW),
                           dcross.reshape(_T * _NOFF, _NH // 2, 2 * _NW))
    return jnp.sum(parts) / jnp.float32(_NQ * _K)
